# Initial kernel scaffold; baseline (speedup 1.0000x reference)
import jax
import jax.numpy as jnp
from jax.experimental import pallas as pl


def kernel(x, edge_index, W1, b1, g1, be1, W2, b2, g2, be2, W3, b3, g3, be3, W4, b4):
    raise NotImplementedError("write your pallas kernel here")


# speedup vs baseline: 17.1969x; 17.1969x over previous
import functools

import jax
import jax.numpy as jnp
from jax import lax
from jax.experimental import pallas as pl
from jax.experimental.pallas import tpu as pltpu
from jax.experimental.pallas import tpu_sc as plsc

N = 10000
E = 160000
C = 256
HALF = 128
NC = 2
NS = 16
RPT = N // NS

EPT = E // NS
K = 80
NCHUNK = EPT // K
NSLOT = 3

EPH = E // (NC * NS)
KH = 40
NCHUNK_H = EPH // KH
HW = 16

BR = 1000
NB = N // BR

_sc_mesh = plsc.VectorSubcoreMesh(core_axis_name="c", subcore_axis_name="s")
_sc_params = pltpu.CompilerParams(use_tc_tiling_on_sc=False)
_tc_params = pltpu.CompilerParams(dimension_semantics=("arbitrary",))


@functools.partial(
    pl.kernel,
    out_type=jax.ShapeDtypeStruct((NC * N, HW), jnp.float32),
    mesh=_sc_mesh,
    compiler_params=_sc_params,
    scratch_types=[
        pltpu.VMEM((NCHUNK_H, KH), jnp.int32),
        pltpu.VMEM((KH, HW), jnp.float32),
        pltpu.VMEM_SHARED((N, HW), jnp.float32),
    ],
)
def _sc_hist(dst_hbm, zeros_hbm, ones_hbm, out_hbm, idx_v, ones_v, shared):
    cid = lax.axis_index("c")
    sid = lax.axis_index("s")
    wid = cid * NS + sid
    rbase = sid * RPT
    pltpu.sync_copy(zeros_hbm.at[pl.ds(rbase, RPT)], shared.at[pl.ds(rbase, RPT)])
    pltpu.sync_copy(ones_hbm, ones_v)
    pltpu.sync_copy(dst_hbm.at[wid], idx_v)
    plsc.subcore_barrier()

    def body(g, carry):
        pltpu.sync_copy(ones_v, shared.at[idx_v.at[g]], add=True)
        return carry

    lax.fori_loop(0, NCHUNK_H, body, 0)
    plsc.subcore_barrier()
    pltpu.sync_copy(shared.at[pl.ds(rbase, RPT)],
                    out_hbm.at[pl.ds(cid * N + rbase, RPT)])


@functools.partial(
    pl.kernel,
    out_type=jax.ShapeDtypeStruct((NC * N, HALF), jnp.float32),
    mesh=_sc_mesh,
    compiler_params=_sc_params,
    scratch_types=[
        pltpu.VMEM((NCHUNK, K), jnp.int32),
        pltpu.VMEM((NCHUNK, K), jnp.int32),
        [pltpu.VMEM((K, HALF), jnp.float32)] * NSLOT,
        pltpu.VMEM_SHARED((N, HALF), jnp.float32),
        [pltpu.SemaphoreType.DMA] * NSLOT,
        [pltpu.SemaphoreType.DMA] * NSLOT,
    ],
)
def _sc_prop(src_hbm, dst_hbm, hp_hbm, out_hbm, idx_s, idx_d, rows4, shared,
             gsem4, ssem4):
    cid = lax.axis_index("c")
    sid = lax.axis_index("s")
    rbase = sid * RPT
    pltpu.sync_copy(hp_hbm.at[pl.ds(cid * N + rbase, RPT)],
                    shared.at[pl.ds(rbase, RPT)])
    wid = cid * NS + sid
    pltpu.sync_copy(src_hbm.at[wid], idx_s)
    pltpu.sync_copy(dst_hbm.at[sid], idx_d)
    plsc.subcore_barrier()

    def _gather(g, j):
        pltpu.async_copy(hp_hbm.at[idx_s.at[g]], rows4[j], gsem4[j])

    def _wait_gather(g, j):
        pltpu.make_async_copy(hp_hbm.at[idx_s.at[g]], rows4[j], gsem4[j]).wait()

    def _scatter(g, j):
        pltpu.async_copy(rows4[j], shared.at[idx_d.at[g]], ssem4[j], add=True)

    def _wait_scatter(g, j):
        pltpu.make_async_copy(rows4[j], shared.at[idx_d.at[g]], ssem4[j]).wait()

    for j in range(NSLOT - 1):
        _gather(j, j)

    def blk(q, carry):
        for j in range(NSLOT):
            g = NSLOT * q + j
            _wait_gather(g, j)
            _scatter(g, j)
            k = (j + NSLOT - 1) % NSLOT
            nxt = g + NSLOT - 1

            @pl.when(g >= 1)
            def _():
                _wait_scatter(g - 1, k)

            @pl.when(nxt < NCHUNK)
            def _():
                _gather(nxt, k)
        return carry

    lax.fori_loop(0, NCHUNK // NSLOT, blk, 0)
    for g in range(NSLOT * (NCHUNK // NSLOT), NCHUNK):
        _wait_gather(g, g % NSLOT)
        _scatter(g, g % NSLOT)
    for g in range(NCHUNK - NSLOT, NCHUNK):
        _wait_scatter(g, g % NSLOT)
    plsc.subcore_barrier()
    pltpu.sync_copy(shared.at[pl.ds(rbase, RPT)],
                    out_hbm.at[pl.ds(cid * N + rbase, RPT)])


def _dinv_of(hist_ref):
    return lax.rsqrt(hist_ref[0, :, 0:1] + hist_ref[1, :, 0:1] + 1.0)


def _tc_prep_body(hist_ref, x_ref, w_ref, hp_ref):
    dinv = _dinv_of(hist_ref)
    h = jnp.dot(x_ref[...], w_ref[...], preferred_element_type=jnp.float32) * dinv
    hp_ref[0] = h[:, :HALF]
    hp_ref[1] = h[:, HALF:]


def _tc_layer_body(acc_ref, hist_ref, res_ref, b_ref, g_ref, be_ref, w_ref,
                   y_ref, hp_ref, z_scr, st_scr, *, has_res, has_y=True):
    p = pl.program_id(0)
    i = pl.program_id(1)
    dinv = _dinv_of(hist_ref)

    @pl.when(p == 0)
    def _():
        z = (jnp.concatenate([acc_ref[0], acc_ref[1]], axis=1) * dinv
             + b_ref[...])
        z_scr[pl.ds(i * BR, BR), :] = z
        s1 = jnp.sum(z, axis=0, keepdims=True)
        s2 = jnp.sum(z * z, axis=0, keepdims=True)
        upd = jnp.concatenate([s1, s2, jnp.zeros((6, C), jnp.float32)], axis=0)
        st_scr[...] = jnp.where(i == 0, upd, st_scr[...] + upd)

    @pl.when(p == 1)
    def _():
        inv_n = 1.0 / N
        mu = st_scr[0:1] * inv_n
        var = st_scr[1:2] * inv_n - mu * mu
        z = z_scr[pl.ds(i * BR, BR), :]
        zn = g_ref[...] * (z - mu) * lax.rsqrt(var + 1e-5) + be_ref[...]
        if has_res:
            zn = zn + res_ref[...]
        y = jnp.maximum(zn, 0.0)
        if has_y:
            y_ref[...] = y
        h = jnp.dot(y, w_ref[...], preferred_element_type=jnp.float32) * dinv
        hp_ref[0] = h[:, :HALF]
        hp_ref[1] = h[:, HALF:]


def _tc_final_body(acc_ref, hist_ref, b_ref, out_ref):
    dinv = _dinv_of(hist_ref)
    out_ref[...] = (jnp.concatenate([acc_ref[0], acc_ref[1]], axis=1) * dinv
                    + b_ref[...])


_hist_spec = pl.BlockSpec((2, BR, HW), lambda i: (0, i, 0))
_acc_spec = pl.BlockSpec((2, BR, HALF), lambda i: (0, i, 0))
_row_spec = pl.BlockSpec((BR, C), lambda i: (i, 0))
_w_spec = pl.BlockSpec((C, C), lambda i: (0, 0))
_vec_spec = pl.BlockSpec((1, C), lambda i: (0, 0))
_st_spec = pl.BlockSpec((8, C), lambda i: (0, 0))
_hp_out_spec = pl.BlockSpec((2, BR, HALF), lambda i: (0, i, 0))

_tc_prep = pl.pallas_call(
    _tc_prep_body,
    grid=(NB,),
    in_specs=[_hist_spec, _row_spec, _w_spec],
    out_specs=_hp_out_spec,
    out_shape=jax.ShapeDtypeStruct((2, N, HALF), jnp.float32),
    compiler_params=_tc_params,
)

_tc2_params = pltpu.CompilerParams(
    dimension_semantics=("arbitrary", "arbitrary"))
_acc2_spec = pl.BlockSpec(
    (2, BR, HALF), lambda p, i: (0, jnp.where(p == 0, i, 0), 0))
_hist2_spec = pl.BlockSpec((2, BR, HW), lambda p, i: (0, i, 0))
_res2_spec = pl.BlockSpec((BR, C), lambda p, i: (jnp.where(p == 1, i, 0), 0))
_vec2_spec = pl.BlockSpec((1, C), lambda p, i: (0, 0))
_w2_spec = pl.BlockSpec((C, C), lambda p, i: (0, 0))
_hp2_out_spec = pl.BlockSpec(
    (2, BR, HALF), lambda p, i: (0, jnp.where(p == 1, i, 0), 0))
_layer_out_shape = [jax.ShapeDtypeStruct((N, C), jnp.float32),
                    jax.ShapeDtypeStruct((2, N, HALF), jnp.float32)]
_layer_scratch = [pltpu.VMEM((N, C), jnp.float32),
                  pltpu.VMEM((8, C), jnp.float32)]

_tc_layer_res = pl.pallas_call(
    functools.partial(_tc_layer_body, has_res=True),
    grid=(2, NB),
    in_specs=[_acc2_spec, _hist2_spec, _res2_spec, _vec2_spec, _vec2_spec,
              _vec2_spec, _w2_spec],
    out_specs=[_res2_spec, _hp2_out_spec],
    out_shape=_layer_out_shape,
    scratch_shapes=_layer_scratch,
    compiler_params=_tc2_params,
)


def _tc_layer_nores_body(acc_ref, hist_ref, b_ref, g_ref, be_ref, w_ref,
                         y_ref, hp_ref, z_scr, st_scr):
    _tc_layer_body(acc_ref, hist_ref, None, b_ref, g_ref, be_ref, w_ref,
                   y_ref, hp_ref, z_scr, st_scr, has_res=False)


_tc_layer_nores = pl.pallas_call(
    _tc_layer_nores_body,
    grid=(2, NB),
    in_specs=[_acc2_spec, _hist2_spec, _vec2_spec, _vec2_spec, _vec2_spec,
              _w2_spec],
    out_specs=[_res2_spec, _hp2_out_spec],
    out_shape=_layer_out_shape,
    scratch_shapes=_layer_scratch,
    compiler_params=_tc2_params,
)

def _tc_layer_noy_body(acc_ref, hist_ref, res_ref, b_ref, g_ref, be_ref,
                       w_ref, hp_ref, z_scr, st_scr):
    _tc_layer_body(acc_ref, hist_ref, res_ref, b_ref, g_ref, be_ref, w_ref,
                   None, hp_ref, z_scr, st_scr, has_res=True, has_y=False)


_tc_layer_noy = pl.pallas_call(
    _tc_layer_noy_body,
    grid=(2, NB),
    in_specs=[_acc2_spec, _hist2_spec, _res2_spec, _vec2_spec, _vec2_spec,
              _vec2_spec, _w2_spec],
    out_specs=[_hp2_out_spec],
    out_shape=[jax.ShapeDtypeStruct((2, N, HALF), jnp.float32)],
    scratch_shapes=_layer_scratch,
    compiler_params=_tc2_params,
)

_tc_final = pl.pallas_call(
    _tc_final_body,
    grid=(NB,),
    in_specs=[_acc_spec, _hist_spec, _vec_spec],
    out_specs=pl.BlockSpec((BR, C), lambda i: (i, 0)),
    out_shape=jax.ShapeDtypeStruct((N, C), jnp.float32),
    compiler_params=_tc_params,
)


def kernel(x, edge_index, W1, b1, g1, be1, W2, b2, g2, be2, W3, b3, g3, be3,
           W4, b4):
    ei = edge_index.astype(jnp.int32)
    src, dst = ei[0], ei[1]

    src_pre = jnp.stack([src, src + N]).reshape(NC * NS, NCHUNK, K)
    dst_prop = dst.reshape(NS, NCHUNK, K)
    dst_hist = dst.reshape(NC * NS, NCHUNK_H, KH)

    zeros_hw = jnp.zeros((N, HW), jnp.float32)
    ones_kh = jnp.ones((KH, HW), jnp.float32)

    hist = _sc_hist(dst_hist, zeros_hw, ones_kh).reshape(NC, N, HW)

    b1r, b2r, b4r = b1.reshape(1, C), b2.reshape(1, C), b4.reshape(1, C)
    b3r = b3.reshape(1, C)
    g1r, g2r, g3r = g1.reshape(1, C), g2.reshape(1, C), g3.reshape(1, C)
    be1r, be2r, be3r = be1.reshape(1, C), be2.reshape(1, C), be3.reshape(1, C)

    hp1 = _tc_prep(hist, x, W1)
    acc1 = _sc_prop(src_pre, dst_prop, hp1.reshape(NC * N, HALF))
    y1, hp2 = _tc_layer_nores(acc1.reshape(NC, N, HALF), hist, b1r, g1r, be1r,
                              W2)

    acc2 = _sc_prop(src_pre, dst_prop, hp2.reshape(NC * N, HALF))
    y2, hp3 = _tc_layer_res(acc2.reshape(NC, N, HALF), hist, y1, b2r, g2r,
                            be2r, W3)

    acc3 = _sc_prop(src_pre, dst_prop, hp3.reshape(NC * N, HALF))
    (hp4,) = _tc_layer_noy(acc3.reshape(NC, N, HALF), hist, y2, b3r, g3r,
                           be3r, W4)

    acc4 = _sc_prop(src_pre, dst_prop, hp4.reshape(NC * N, HALF))
    return _tc_final(acc4.reshape(NC, N, HALF), hist, b4r)


# speedup vs baseline: 17.7487x; 1.0321x over previous
import functools

import jax
import jax.numpy as jnp
from jax import lax
from jax.experimental import pallas as pl
from jax.experimental.pallas import tpu as pltpu
from jax.experimental.pallas import tpu_sc as plsc

N = 10000
E = 160000
C = 256
HALF = 128
NC = 2
NS = 16
RPT = N // NS

EPT = E // NS
K = 80
NCHUNK = EPT // K
NSLOT = 3

EPH = E // (NC * NS)
KH = 40
NCHUNK_H = EPH // KH
HW = 16

BR = 2000
NB = N // BR

_sc_mesh = plsc.VectorSubcoreMesh(core_axis_name="c", subcore_axis_name="s")
_sc_params = pltpu.CompilerParams(use_tc_tiling_on_sc=False)
_tc_params = pltpu.CompilerParams(dimension_semantics=("arbitrary",))


@functools.partial(
    pl.kernel,
    out_type=jax.ShapeDtypeStruct((NC * N, HW), jnp.float32),
    mesh=_sc_mesh,
    compiler_params=_sc_params,
    scratch_types=[
        pltpu.VMEM((NCHUNK_H, KH), jnp.int32),
        pltpu.VMEM((KH, HW), jnp.float32),
        pltpu.VMEM_SHARED((N, HW), jnp.float32),
    ],
)
def _sc_hist(dst_hbm, zeros_hbm, ones_hbm, out_hbm, idx_v, ones_v, shared):
    cid = lax.axis_index("c")
    sid = lax.axis_index("s")
    wid = cid * NS + sid
    rbase = sid * RPT
    pltpu.sync_copy(zeros_hbm.at[pl.ds(rbase, RPT)], shared.at[pl.ds(rbase, RPT)])
    pltpu.sync_copy(ones_hbm, ones_v)
    pltpu.sync_copy(dst_hbm.at[wid], idx_v)
    plsc.subcore_barrier()

    def body(g, carry):
        pltpu.sync_copy(ones_v, shared.at[idx_v.at[g]], add=True)
        return carry

    lax.fori_loop(0, NCHUNK_H, body, 0)
    plsc.subcore_barrier()
    pltpu.sync_copy(shared.at[pl.ds(rbase, RPT)],
                    out_hbm.at[pl.ds(cid * N + rbase, RPT)])


@functools.partial(
    pl.kernel,
    out_type=jax.ShapeDtypeStruct((NC * N, HALF), jnp.float32),
    mesh=_sc_mesh,
    compiler_params=_sc_params,
    scratch_types=[
        pltpu.VMEM((NCHUNK, K), jnp.int32),
        pltpu.VMEM((NCHUNK, K), jnp.int32),
        [pltpu.VMEM((K, HALF), jnp.float32)] * NSLOT,
        pltpu.VMEM_SHARED((N, HALF), jnp.float32),
        [pltpu.SemaphoreType.DMA] * NSLOT,
        [pltpu.SemaphoreType.DMA] * NSLOT,
    ],
)
def _sc_prop(src_hbm, dst_hbm, hp_hbm, out_hbm, idx_s, idx_d, rows4, shared,
             gsem4, ssem4):
    cid = lax.axis_index("c")
    sid = lax.axis_index("s")
    rbase = sid * RPT
    pltpu.sync_copy(hp_hbm.at[pl.ds(cid * N + rbase, RPT)],
                    shared.at[pl.ds(rbase, RPT)])
    wid = cid * NS + sid
    pltpu.sync_copy(src_hbm.at[wid], idx_s)
    pltpu.sync_copy(dst_hbm.at[sid], idx_d)
    plsc.subcore_barrier()

    def _gather(g, j):
        pltpu.async_copy(hp_hbm.at[idx_s.at[g]], rows4[j], gsem4[j])

    def _wait_gather(g, j):
        pltpu.make_async_copy(hp_hbm.at[idx_s.at[g]], rows4[j], gsem4[j]).wait()

    def _scatter(g, j):
        pltpu.async_copy(rows4[j], shared.at[idx_d.at[g]], ssem4[j], add=True)

    def _wait_scatter(g, j):
        pltpu.make_async_copy(rows4[j], shared.at[idx_d.at[g]], ssem4[j]).wait()

    for j in range(NSLOT - 1):
        _gather(j, j)

    def blk(q, carry):
        for j in range(NSLOT):
            g = NSLOT * q + j
            _wait_gather(g, j)
            _scatter(g, j)
            k = (j + NSLOT - 1) % NSLOT
            nxt = g + NSLOT - 1

            @pl.when(g >= 1)
            def _():
                _wait_scatter(g - 1, k)

            @pl.when(nxt < NCHUNK)
            def _():
                _gather(nxt, k)
        return carry

    lax.fori_loop(0, NCHUNK // NSLOT, blk, 0)
    for g in range(NSLOT * (NCHUNK // NSLOT), NCHUNK):
        _wait_gather(g, g % NSLOT)
        _scatter(g, g % NSLOT)
    for g in range(NCHUNK - NSLOT, NCHUNK):
        _wait_scatter(g, g % NSLOT)
    plsc.subcore_barrier()
    pltpu.sync_copy(shared.at[pl.ds(rbase, RPT)],
                    out_hbm.at[pl.ds(cid * N + rbase, RPT)])


def _dinv_of(hist_ref):
    return lax.rsqrt(hist_ref[0, :, 0:1] + hist_ref[1, :, 0:1] + 1.0)


def _tc_prep_body(hist_ref, x_ref, w_ref, hp_ref):
    dinv = _dinv_of(hist_ref)
    h = jnp.dot(x_ref[...], w_ref[...], preferred_element_type=jnp.float32) * dinv
    hp_ref[0] = h[:, :HALF]
    hp_ref[1] = h[:, HALF:]


def _tc_layer_body(acc_ref, hist_ref, res_ref, b_ref, g_ref, be_ref, w_ref,
                   y_ref, hp_ref, z_scr, st_scr, *, has_res, has_y=True):
    p = pl.program_id(0)
    i = pl.program_id(1)
    dinv = _dinv_of(hist_ref)

    @pl.when(p == 0)
    def _():
        z = (jnp.concatenate([acc_ref[0], acc_ref[1]], axis=1) * dinv
             + b_ref[...])
        z_scr[pl.ds(i * BR, BR), :] = z
        s1 = jnp.sum(z, axis=0, keepdims=True)
        s2 = jnp.sum(z * z, axis=0, keepdims=True)
        upd = jnp.concatenate([s1, s2, jnp.zeros((6, C), jnp.float32)], axis=0)
        st_scr[...] = jnp.where(i == 0, upd, st_scr[...] + upd)

    @pl.when(p == 1)
    def _():
        inv_n = 1.0 / N
        mu = st_scr[0:1] * inv_n
        var = st_scr[1:2] * inv_n - mu * mu
        z = z_scr[pl.ds(i * BR, BR), :]
        zn = g_ref[...] * (z - mu) * lax.rsqrt(var + 1e-5) + be_ref[...]
        if has_res:
            zn = zn + res_ref[...]
        y = jnp.maximum(zn, 0.0)
        if has_y:
            y_ref[...] = y
        h = jnp.dot(y, w_ref[...], preferred_element_type=jnp.float32) * dinv
        hp_ref[0] = h[:, :HALF]
        hp_ref[1] = h[:, HALF:]


def _tc_final_body(acc_ref, hist_ref, b_ref, out_ref):
    dinv = _dinv_of(hist_ref)
    out_ref[...] = (jnp.concatenate([acc_ref[0], acc_ref[1]], axis=1) * dinv
                    + b_ref[...])


_hist_spec = pl.BlockSpec((2, BR, HW), lambda i: (0, i, 0))
_acc_spec = pl.BlockSpec((2, BR, HALF), lambda i: (0, i, 0))
_row_spec = pl.BlockSpec((BR, C), lambda i: (i, 0))
_w_spec = pl.BlockSpec((C, C), lambda i: (0, 0))
_vec_spec = pl.BlockSpec((1, C), lambda i: (0, 0))
_st_spec = pl.BlockSpec((8, C), lambda i: (0, 0))
_hp_out_spec = pl.BlockSpec((2, BR, HALF), lambda i: (0, i, 0))

_tc_prep = pl.pallas_call(
    _tc_prep_body,
    grid=(NB,),
    in_specs=[_hist_spec, _row_spec, _w_spec],
    out_specs=_hp_out_spec,
    out_shape=jax.ShapeDtypeStruct((2, N, HALF), jnp.float32),
    compiler_params=_tc_params,
)

_tc2_params = pltpu.CompilerParams(
    dimension_semantics=("arbitrary", "arbitrary"))
_acc2_spec = pl.BlockSpec(
    (2, BR, HALF), lambda p, i: (0, jnp.where(p == 0, i, 0), 0))
_hist2_spec = pl.BlockSpec((2, BR, HW), lambda p, i: (0, i, 0))
_res2_spec = pl.BlockSpec((BR, C), lambda p, i: (jnp.where(p == 1, i, 0), 0))
_vec2_spec = pl.BlockSpec((1, C), lambda p, i: (0, 0))
_w2_spec = pl.BlockSpec((C, C), lambda p, i: (0, 0))
_hp2_out_spec = pl.BlockSpec(
    (2, BR, HALF), lambda p, i: (0, jnp.where(p == 1, i, 0), 0))
_layer_out_shape = [jax.ShapeDtypeStruct((N, C), jnp.float32),
                    jax.ShapeDtypeStruct((2, N, HALF), jnp.float32)]
_layer_scratch = [pltpu.VMEM((N, C), jnp.float32),
                  pltpu.VMEM((8, C), jnp.float32)]

_tc_layer_res = pl.pallas_call(
    functools.partial(_tc_layer_body, has_res=True),
    grid=(2, NB),
    in_specs=[_acc2_spec, _hist2_spec, _res2_spec, _vec2_spec, _vec2_spec,
              _vec2_spec, _w2_spec],
    out_specs=[_res2_spec, _hp2_out_spec],
    out_shape=_layer_out_shape,
    scratch_shapes=_layer_scratch,
    compiler_params=_tc2_params,
)


def _tc_layer_nores_body(acc_ref, hist_ref, b_ref, g_ref, be_ref, w_ref,
                         y_ref, hp_ref, z_scr, st_scr):
    _tc_layer_body(acc_ref, hist_ref, None, b_ref, g_ref, be_ref, w_ref,
                   y_ref, hp_ref, z_scr, st_scr, has_res=False)


_tc_layer_nores = pl.pallas_call(
    _tc_layer_nores_body,
    grid=(2, NB),
    in_specs=[_acc2_spec, _hist2_spec, _vec2_spec, _vec2_spec, _vec2_spec,
              _w2_spec],
    out_specs=[_res2_spec, _hp2_out_spec],
    out_shape=_layer_out_shape,
    scratch_shapes=_layer_scratch,
    compiler_params=_tc2_params,
)

def _tc_layer_noy_body(acc_ref, hist_ref, res_ref, b_ref, g_ref, be_ref,
                       w_ref, hp_ref, z_scr, st_scr):
    _tc_layer_body(acc_ref, hist_ref, res_ref, b_ref, g_ref, be_ref, w_ref,
                   None, hp_ref, z_scr, st_scr, has_res=True, has_y=False)


_tc_layer_noy = pl.pallas_call(
    _tc_layer_noy_body,
    grid=(2, NB),
    in_specs=[_acc2_spec, _hist2_spec, _res2_spec, _vec2_spec, _vec2_spec,
              _vec2_spec, _w2_spec],
    out_specs=[_hp2_out_spec],
    out_shape=[jax.ShapeDtypeStruct((2, N, HALF), jnp.float32)],
    scratch_shapes=_layer_scratch,
    compiler_params=_tc2_params,
)

_tc_final = pl.pallas_call(
    _tc_final_body,
    grid=(NB,),
    in_specs=[_acc_spec, _hist_spec, _vec_spec],
    out_specs=pl.BlockSpec((BR, C), lambda i: (i, 0)),
    out_shape=jax.ShapeDtypeStruct((N, C), jnp.float32),
    compiler_params=_tc_params,
)


def kernel(x, edge_index, W1, b1, g1, be1, W2, b2, g2, be2, W3, b3, g3, be3,
           W4, b4):
    ei = edge_index.astype(jnp.int32)
    src, dst = ei[0], ei[1]

    src_pre = jnp.stack([src, src + N]).reshape(NC * NS, NCHUNK, K)
    dst_prop = dst.reshape(NS, NCHUNK, K)
    dst_hist = dst.reshape(NC * NS, NCHUNK_H, KH)

    zeros_hw = jnp.zeros((N, HW), jnp.float32)
    ones_kh = jnp.ones((KH, HW), jnp.float32)

    hist = _sc_hist(dst_hist, zeros_hw, ones_kh).reshape(NC, N, HW)

    b1r, b2r, b4r = b1.reshape(1, C), b2.reshape(1, C), b4.reshape(1, C)
    b3r = b3.reshape(1, C)
    g1r, g2r, g3r = g1.reshape(1, C), g2.reshape(1, C), g3.reshape(1, C)
    be1r, be2r, be3r = be1.reshape(1, C), be2.reshape(1, C), be3.reshape(1, C)

    hp1 = _tc_prep(hist, x, W1)
    acc1 = _sc_prop(src_pre, dst_prop, hp1.reshape(NC * N, HALF))
    y1, hp2 = _tc_layer_nores(acc1.reshape(NC, N, HALF), hist, b1r, g1r, be1r,
                              W2)

    acc2 = _sc_prop(src_pre, dst_prop, hp2.reshape(NC * N, HALF))
    y2, hp3 = _tc_layer_res(acc2.reshape(NC, N, HALF), hist, y1, b2r, g2r,
                            be2r, W3)

    acc3 = _sc_prop(src_pre, dst_prop, hp3.reshape(NC * N, HALF))
    (hp4,) = _tc_layer_noy(acc3.reshape(NC, N, HALF), hist, y2, b3r, g3r,
                           be3r, W4)

    acc4 = _sc_prop(src_pre, dst_prop, hp4.reshape(NC * N, HALF))
    return _tc_final(acc4.reshape(NC, N, HALF), hist, b4r)


# speedup vs baseline: 17.9141x; 1.0093x over previous
import functools

import jax
import jax.numpy as jnp
from jax import lax
from jax.experimental import pallas as pl
from jax.experimental.pallas import tpu as pltpu
from jax.experimental.pallas import tpu_sc as plsc

N = 10000
E = 160000
C = 256
HALF = 128
NC = 2
NS = 16
RPT = N // NS

EPT = E // NS
K = 80
NCHUNK = EPT // K
NSLOT = 3

EPH = E // (NC * NS)
KH = 40
NCHUNK_H = EPH // KH
HW = 16

BR = 2000
NB = N // BR

_sc_mesh = plsc.VectorSubcoreMesh(core_axis_name="c", subcore_axis_name="s")
_sc_params = pltpu.CompilerParams(use_tc_tiling_on_sc=False)
_tc_params = pltpu.CompilerParams(dimension_semantics=("arbitrary",))


@functools.partial(
    pl.kernel,
    out_type=jax.ShapeDtypeStruct((NC * N, HW), jnp.float32),
    mesh=_sc_mesh,
    compiler_params=_sc_params,
    scratch_types=[
        pltpu.VMEM((NCHUNK_H, KH), jnp.int32),
        pltpu.VMEM((KH, HW), jnp.float32),
        pltpu.VMEM_SHARED((N, HW), jnp.float32),
    ],
)
def _sc_hist(dst_hbm, zeros_hbm, ones_hbm, out_hbm, idx_v, ones_v, shared):
    cid = lax.axis_index("c")
    sid = lax.axis_index("s")
    wid = cid * NS + sid
    rbase = sid * RPT
    pltpu.sync_copy(zeros_hbm.at[pl.ds(rbase, RPT)], shared.at[pl.ds(rbase, RPT)])
    pltpu.sync_copy(ones_hbm, ones_v)
    pltpu.sync_copy(dst_hbm.at[wid], idx_v)
    plsc.subcore_barrier()

    def body(g, carry):
        pltpu.sync_copy(ones_v, shared.at[idx_v.at[g]], add=True)
        return carry

    lax.fori_loop(0, NCHUNK_H, body, 0)
    plsc.subcore_barrier()
    pltpu.sync_copy(shared.at[pl.ds(rbase, RPT)],
                    out_hbm.at[pl.ds(cid * N + rbase, RPT)])


@functools.partial(
    pl.kernel,
    out_type=jax.ShapeDtypeStruct((NC * N, HALF), jnp.float32),
    mesh=_sc_mesh,
    compiler_params=_sc_params,
    scratch_types=[
        pltpu.VMEM((NCHUNK, K), jnp.int32),
        pltpu.VMEM((NCHUNK, K), jnp.int32),
        [pltpu.VMEM((K, HALF), jnp.float32)] * NSLOT,
        pltpu.VMEM_SHARED((N, HALF), jnp.float32),
        [pltpu.SemaphoreType.DMA] * NSLOT,
        [pltpu.SemaphoreType.DMA] * NSLOT,
    ],
)
def _sc_prop(src_hbm, dst_hbm, hp_hbm, out_hbm, idx_s, idx_d, rows4, shared,
             gsem4, ssem4):
    cid = lax.axis_index("c")
    sid = lax.axis_index("s")
    rbase = sid * RPT
    wid = cid * NS + sid
    pltpu.sync_copy(src_hbm.at[wid], idx_s)
    pltpu.sync_copy(dst_hbm.at[sid], idx_d)

    def _gather(g, j):
        pltpu.async_copy(hp_hbm.at[idx_s.at[g]], rows4[j], gsem4[j])

    def _wait_gather(g, j):
        pltpu.make_async_copy(hp_hbm.at[idx_s.at[g]], rows4[j], gsem4[j]).wait()

    def _scatter(g, j):
        pltpu.async_copy(rows4[j], shared.at[idx_d.at[g]], ssem4[j], add=True)

    def _wait_scatter(g, j):
        pltpu.make_async_copy(rows4[j], shared.at[idx_d.at[g]], ssem4[j]).wait()

    for j in range(NSLOT - 1):
        _gather(j, j)
    pltpu.sync_copy(hp_hbm.at[pl.ds(cid * N + rbase, RPT)],
                    shared.at[pl.ds(rbase, RPT)])
    plsc.subcore_barrier()

    def blk(q, carry):
        for j in range(NSLOT):
            g = NSLOT * q + j
            _wait_gather(g, j)
            _scatter(g, j)
            k = (j + NSLOT - 1) % NSLOT
            nxt = g + NSLOT - 1

            @pl.when(g >= 1)
            def _():
                _wait_scatter(g - 1, k)

            @pl.when(nxt < NCHUNK)
            def _():
                _gather(nxt, k)
        return carry

    lax.fori_loop(0, NCHUNK // NSLOT, blk, 0)
    for g in range(NSLOT * (NCHUNK // NSLOT), NCHUNK):
        _wait_gather(g, g % NSLOT)
        _scatter(g, g % NSLOT)
    for g in range(NCHUNK - NSLOT, NCHUNK):
        _wait_scatter(g, g % NSLOT)
    plsc.subcore_barrier()
    pltpu.sync_copy(shared.at[pl.ds(rbase, RPT)],
                    out_hbm.at[pl.ds(cid * N + rbase, RPT)])


def _dinv_of(hist_ref):
    return lax.rsqrt(hist_ref[0, :, 0:1] + hist_ref[1, :, 0:1] + 1.0)


def _tc_prep_body(hist_ref, x_ref, w_ref, hp_ref):
    dinv = _dinv_of(hist_ref)
    h = jnp.dot(x_ref[...], w_ref[...], preferred_element_type=jnp.float32) * dinv
    hp_ref[0] = h[:, :HALF]
    hp_ref[1] = h[:, HALF:]


def _tc_layer_body(acc_ref, hist_ref, res_ref, b_ref, g_ref, be_ref, w_ref,
                   y_ref, hp_ref, z_scr, st_scr, *, has_res, has_y=True):
    p = pl.program_id(0)
    i = pl.program_id(1)
    dinv = _dinv_of(hist_ref)

    @pl.when(p == 0)
    def _():
        z = (jnp.concatenate([acc_ref[0], acc_ref[1]], axis=1) * dinv
             + b_ref[...])
        z_scr[pl.ds(i * BR, BR), :] = z
        s1 = jnp.sum(z, axis=0, keepdims=True)
        s2 = jnp.sum(z * z, axis=0, keepdims=True)
        upd = jnp.concatenate([s1, s2, jnp.zeros((6, C), jnp.float32)], axis=0)
        st_scr[...] = jnp.where(i == 0, upd, st_scr[...] + upd)

    @pl.when(p == 1)
    def _():
        inv_n = 1.0 / N
        mu = st_scr[0:1] * inv_n
        var = st_scr[1:2] * inv_n - mu * mu
        z = z_scr[pl.ds(i * BR, BR), :]
        zn = g_ref[...] * (z - mu) * lax.rsqrt(var + 1e-5) + be_ref[...]
        if has_res:
            zn = zn + res_ref[...]
        y = jnp.maximum(zn, 0.0)
        if has_y:
            y_ref[...] = y
        h = jnp.dot(y, w_ref[...], preferred_element_type=jnp.float32) * dinv
        hp_ref[0] = h[:, :HALF]
        hp_ref[1] = h[:, HALF:]


def _tc_final_body(acc_ref, hist_ref, b_ref, out_ref):
    dinv = _dinv_of(hist_ref)
    out_ref[...] = (jnp.concatenate([acc_ref[0], acc_ref[1]], axis=1) * dinv
                    + b_ref[...])


_hist_spec = pl.BlockSpec((2, BR, HW), lambda i: (0, i, 0))
_acc_spec = pl.BlockSpec((2, BR, HALF), lambda i: (0, i, 0))
_row_spec = pl.BlockSpec((BR, C), lambda i: (i, 0))
_w_spec = pl.BlockSpec((C, C), lambda i: (0, 0))
_vec_spec = pl.BlockSpec((1, C), lambda i: (0, 0))
_st_spec = pl.BlockSpec((8, C), lambda i: (0, 0))
_hp_out_spec = pl.BlockSpec((2, BR, HALF), lambda i: (0, i, 0))

_tc_prep = pl.pallas_call(
    _tc_prep_body,
    grid=(NB,),
    in_specs=[_hist_spec, _row_spec, _w_spec],
    out_specs=_hp_out_spec,
    out_shape=jax.ShapeDtypeStruct((2, N, HALF), jnp.float32),
    compiler_params=_tc_params,
)

_tc2_params = pltpu.CompilerParams(
    dimension_semantics=("arbitrary", "arbitrary"))
_acc2_spec = pl.BlockSpec(
    (2, BR, HALF), lambda p, i: (0, jnp.where(p == 0, i, 0), 0))
_hist2_spec = pl.BlockSpec((2, BR, HW), lambda p, i: (0, i, 0))
_res2_spec = pl.BlockSpec((BR, C), lambda p, i: (jnp.where(p == 1, i, 0), 0))
_vec2_spec = pl.BlockSpec((1, C), lambda p, i: (0, 0))
_w2_spec = pl.BlockSpec((C, C), lambda p, i: (0, 0))
_hp2_out_spec = pl.BlockSpec(
    (2, BR, HALF), lambda p, i: (0, jnp.where(p == 1, i, 0), 0))
_layer_out_shape = [jax.ShapeDtypeStruct((N, C), jnp.float32),
                    jax.ShapeDtypeStruct((2, N, HALF), jnp.float32)]
_layer_scratch = [pltpu.VMEM((N, C), jnp.float32),
                  pltpu.VMEM((8, C), jnp.float32)]

_tc_layer_res = pl.pallas_call(
    functools.partial(_tc_layer_body, has_res=True),
    grid=(2, NB),
    in_specs=[_acc2_spec, _hist2_spec, _res2_spec, _vec2_spec, _vec2_spec,
              _vec2_spec, _w2_spec],
    out_specs=[_res2_spec, _hp2_out_spec],
    out_shape=_layer_out_shape,
    scratch_shapes=_layer_scratch,
    compiler_params=_tc2_params,
)


def _tc_layer_nores_body(acc_ref, hist_ref, b_ref, g_ref, be_ref, w_ref,
                         y_ref, hp_ref, z_scr, st_scr):
    _tc_layer_body(acc_ref, hist_ref, None, b_ref, g_ref, be_ref, w_ref,
                   y_ref, hp_ref, z_scr, st_scr, has_res=False)


_tc_layer_nores = pl.pallas_call(
    _tc_layer_nores_body,
    grid=(2, NB),
    in_specs=[_acc2_spec, _hist2_spec, _vec2_spec, _vec2_spec, _vec2_spec,
              _w2_spec],
    out_specs=[_res2_spec, _hp2_out_spec],
    out_shape=_layer_out_shape,
    scratch_shapes=_layer_scratch,
    compiler_params=_tc2_params,
)

def _tc_layer_noy_body(acc_ref, hist_ref, res_ref, b_ref, g_ref, be_ref,
                       w_ref, hp_ref, z_scr, st_scr):
    _tc_layer_body(acc_ref, hist_ref, res_ref, b_ref, g_ref, be_ref, w_ref,
                   None, hp_ref, z_scr, st_scr, has_res=True, has_y=False)


_tc_layer_noy = pl.pallas_call(
    _tc_layer_noy_body,
    grid=(2, NB),
    in_specs=[_acc2_spec, _hist2_spec, _res2_spec, _vec2_spec, _vec2_spec,
              _vec2_spec, _w2_spec],
    out_specs=[_hp2_out_spec],
    out_shape=[jax.ShapeDtypeStruct((2, N, HALF), jnp.float32)],
    scratch_shapes=_layer_scratch,
    compiler_params=_tc2_params,
)

_tc_final = pl.pallas_call(
    _tc_final_body,
    grid=(NB,),
    in_specs=[_acc_spec, _hist_spec, _vec_spec],
    out_specs=pl.BlockSpec((BR, C), lambda i: (i, 0)),
    out_shape=jax.ShapeDtypeStruct((N, C), jnp.float32),
    compiler_params=_tc_params,
)


def kernel(x, edge_index, W1, b1, g1, be1, W2, b2, g2, be2, W3, b3, g3, be3,
           W4, b4):
    ei = edge_index.astype(jnp.int32)
    src, dst = ei[0], ei[1]

    src_pre = jnp.stack([src, src + N]).reshape(NC * NS, NCHUNK, K)
    dst_prop = dst.reshape(NS, NCHUNK, K)
    dst_hist = dst.reshape(NC * NS, NCHUNK_H, KH)

    zeros_hw = jnp.zeros((N, HW), jnp.float32)
    ones_kh = jnp.ones((KH, HW), jnp.float32)

    hist = _sc_hist(dst_hist, zeros_hw, ones_kh).reshape(NC, N, HW)

    b1r, b2r, b4r = b1.reshape(1, C), b2.reshape(1, C), b4.reshape(1, C)
    b3r = b3.reshape(1, C)
    g1r, g2r, g3r = g1.reshape(1, C), g2.reshape(1, C), g3.reshape(1, C)
    be1r, be2r, be3r = be1.reshape(1, C), be2.reshape(1, C), be3.reshape(1, C)

    hp1 = _tc_prep(hist, x, W1)
    acc1 = _sc_prop(src_pre, dst_prop, hp1.reshape(NC * N, HALF))
    y1, hp2 = _tc_layer_nores(acc1.reshape(NC, N, HALF), hist, b1r, g1r, be1r,
                              W2)

    acc2 = _sc_prop(src_pre, dst_prop, hp2.reshape(NC * N, HALF))
    y2, hp3 = _tc_layer_res(acc2.reshape(NC, N, HALF), hist, y1, b2r, g2r,
                            be2r, W3)

    acc3 = _sc_prop(src_pre, dst_prop, hp3.reshape(NC * N, HALF))
    (hp4,) = _tc_layer_noy(acc3.reshape(NC, N, HALF), hist, y2, b3r, g3r,
                           be3r, W4)

    acc4 = _sc_prop(src_pre, dst_prop, hp4.reshape(NC * N, HALF))
    return _tc_final(acc4.reshape(NC, N, HALF), hist, b4r)
